# Initial kernel scaffold; baseline (speedup 1.0000x reference)
import functools

import jax
import jax.numpy as jnp
from jax.experimental import pallas as pl
from jax.experimental.pallas import tpu as pltpu

B = 4
S = 8192
D = 768
BLK = 512
BLKL = 512
NEG = -3.0e38


def _scores_topk_summary_kernel(x_blk, ws, x_any, wr, br, summary_out,
                                scores_s, rows_s, sem):
    i = pl.program_id(0)
    nsteps = pl.num_programs(0)
    xb = x_blk[...]
    sc = jax.lax.dot_general(xb.reshape(B * BLK, D), ws[...],
                             (((1,), (1,)), ((), ())),
                             preferred_element_type=jnp.float32)
    scores_s[:, pl.ds(i * BLK, BLK)] = sc.reshape(B, BLK)

    @pl.when(i == nsteps - 1)
    def _finish():
        iota = jax.lax.broadcasted_iota(jnp.int32, (1, S), 1)
        for b in range(B):
            row = scores_s[pl.ds(b, 1), :]
            m1 = jnp.max(row)
            i1 = jnp.min(jnp.where(row == m1, iota, S))
            row2 = jnp.where(iota == i1, NEG, row)
            m2 = jnp.max(row2)
            i2 = jnp.min(jnp.where(row2 == m2, iota, S))
            cp1 = pltpu.make_async_copy(
                x_any.at[b].at[pl.ds(i1, 1)], rows_s.at[pl.ds(2 * b, 1)], sem)
            cp1.start()
            cp2 = pltpu.make_async_copy(
                x_any.at[b].at[pl.ds(i2, 1)], rows_s.at[pl.ds(2 * b + 1, 1)], sem)
            cp2.start()
            cp1.wait()
            cp2.wait()
        xmean = rows_s[...].reshape(B, 2, D).mean(axis=1)
        summ = jax.lax.dot_general(xmean, wr[...], (((1,), (1,)), ((), ())),
                                   preferred_element_type=jnp.float32)
        summary_out[...] = summ + br[...]


def _ln_kernel(x_blk, summary, gamma, beta, out):
    h = x_blk[...] + summary[...][:, None, :]
    mu = jnp.mean(h, axis=-1, keepdims=True)
    hc = h - mu
    var = jnp.mean(hc * hc, axis=-1, keepdims=True)
    inv = jax.lax.rsqrt(var + 1e-5)
    out[...] = hc * inv * gamma[...][None, :, :] + beta[...][None, :, :]


@jax.jit
def kernel(x, Wr, br, Ws, bs, gamma, beta):
    del bs
    summary = pl.pallas_call(
        _scores_topk_summary_kernel,
        grid=(S // BLK,),
        in_specs=[
            pl.BlockSpec((B, BLK, D), lambda i: (0, i, 0)),
            pl.BlockSpec((1, D), lambda i: (0, 0)),
            pl.BlockSpec(memory_space=pltpu.ANY),
            pl.BlockSpec((D, D), lambda i: (0, 0)),
            pl.BlockSpec((1, D), lambda i: (0, 0)),
        ],
        out_specs=pl.BlockSpec((B, D), lambda i: (0, 0)),
        out_shape=jax.ShapeDtypeStruct((B, D), jnp.float32),
        scratch_shapes=[
            pltpu.VMEM((B, S), jnp.float32),
            pltpu.VMEM((2 * B, D), jnp.float32),
            pltpu.SemaphoreType.DMA,
        ],
        compiler_params=pltpu.CompilerParams(
            dimension_semantics=("arbitrary",)),
    )(x, Ws, x, Wr, br.reshape(1, D))

    out = pl.pallas_call(
        _ln_kernel,
        grid=(S // BLKL,),
        in_specs=[
            pl.BlockSpec((B, BLKL, D), lambda i: (0, i, 0)),
            pl.BlockSpec((B, D), lambda i: (0, 0)),
            pl.BlockSpec((1, D), lambda i: (0, 0)),
            pl.BlockSpec((1, D), lambda i: (0, 0)),
        ],
        out_specs=pl.BlockSpec((B, BLKL, D), lambda i: (0, i, 0)),
        out_shape=jax.ShapeDtypeStruct((B, S, D), jnp.float32),
        compiler_params=pltpu.CompilerParams(
            dimension_semantics=("arbitrary",)),
    )(x, summary, gamma.reshape(1, D), beta.reshape(1, D))
    return out


# speedup vs baseline: 2.1416x; 2.1416x over previous
import functools

import jax
import jax.numpy as jnp
from jax.experimental import pallas as pl
from jax.experimental.pallas import tpu as pltpu

B = 4
S = 8192
D = 768
BLK = 512
BLKL = 512
NEG = -3.0e38


def _scores_topk_summary_kernel(x_blk, ws, x_any, wr, br, summary_out,
                                scores_s, rows_s, sem):
    i = pl.program_id(0)
    nsteps = pl.num_programs(0)
    xb = x_blk[...].astype(jnp.bfloat16)
    sc = jax.lax.dot_general(xb.reshape(B * BLK, D),
                             ws[...].astype(jnp.bfloat16),
                             (((1,), (1,)), ((), ())),
                             preferred_element_type=jnp.float32)
    scores_s[:, pl.ds(i * BLK, BLK)] = sc[:, 0].reshape(B, BLK)

    @pl.when(i == nsteps - 1)
    def _finish():
        iota = jax.lax.broadcasted_iota(jnp.int32, (1, S), 1)
        for b in range(B):
            row = scores_s[pl.ds(b, 1), :]
            m1 = jnp.max(row)
            i1 = jnp.min(jnp.where(row == m1, iota, S))
            row2 = jnp.where(iota == i1, NEG, row)
            m2 = jnp.max(row2)
            i2 = jnp.min(jnp.where(row2 == m2, iota, S))
            cp1 = pltpu.make_async_copy(
                x_any.at[b].at[pl.ds(i1, 1)], rows_s.at[pl.ds(2 * b, 1)], sem)
            cp1.start()
            cp2 = pltpu.make_async_copy(
                x_any.at[b].at[pl.ds(i2, 1)], rows_s.at[pl.ds(2 * b + 1, 1)], sem)
            cp2.start()
            cp1.wait()
            cp2.wait()
        xmean = rows_s[...].reshape(B, 2, D).mean(axis=1)
        summ = jax.lax.dot_general(xmean, wr[...], (((1,), (1,)), ((), ())),
                                   preferred_element_type=jnp.float32)
        summary_out[...] = summ + br[...]


def _ln_kernel(x_blk, summary, gamma, beta, out):
    h = x_blk[...] + summary[...][:, None, :]
    mu = jnp.mean(h, axis=-1, keepdims=True)
    hc = h - mu
    var = jnp.mean(hc * hc, axis=-1, keepdims=True)
    inv = jax.lax.rsqrt(var + 1e-5)
    out[...] = hc * inv * gamma[...][None, :, :] + beta[...][None, :, :]


@jax.jit
def kernel(x, Wr, br, Ws, bs, gamma, beta):
    del bs
    summary = pl.pallas_call(
        _scores_topk_summary_kernel,
        grid=(S // BLK,),
        in_specs=[
            pl.BlockSpec((B, BLK, D), lambda i: (0, i, 0)),
            pl.BlockSpec((8, D), lambda i: (0, 0)),
            pl.BlockSpec(memory_space=pl.ANY),
            pl.BlockSpec((D, D), lambda i: (0, 0)),
            pl.BlockSpec((1, D), lambda i: (0, 0)),
        ],
        out_specs=pl.BlockSpec((B, D), lambda i: (0, 0)),
        out_shape=jax.ShapeDtypeStruct((B, D), jnp.float32),
        scratch_shapes=[
            pltpu.VMEM((B, S), jnp.float32),
            pltpu.VMEM((2 * B, D), jnp.float32),
            pltpu.SemaphoreType.DMA,
        ],
        compiler_params=pltpu.CompilerParams(
            dimension_semantics=("arbitrary",)),
    )(x, jnp.broadcast_to(Ws, (8, D)), x, Wr, br.reshape(1, D))

    out = pl.pallas_call(
        _ln_kernel,
        grid=(S // BLKL,),
        in_specs=[
            pl.BlockSpec((B, BLKL, D), lambda i: (0, i, 0)),
            pl.BlockSpec((B, D), lambda i: (0, 0)),
            pl.BlockSpec((1, D), lambda i: (0, 0)),
            pl.BlockSpec((1, D), lambda i: (0, 0)),
        ],
        out_specs=pl.BlockSpec((B, BLKL, D), lambda i: (0, i, 0)),
        out_shape=jax.ShapeDtypeStruct((B, S, D), jnp.float32),
        compiler_params=pltpu.CompilerParams(
            dimension_semantics=("arbitrary",)),
    )(x, summary, gamma.reshape(1, D), beta.reshape(1, D))
    return out


# speedup vs baseline: 2.1632x; 1.0101x over previous
import functools

import jax
import jax.numpy as jnp
from jax.experimental import pallas as pl
from jax.experimental.pallas import tpu as pltpu

B = 4
S = 8192
D = 768
BLK = 1024
BLKL = 512
NEG = -3.0e38


def _scores_topk_summary_kernel(x_blk, ws, x_any, wr, br, summary_out,
                                scores_s, rows_s, sem):
    i = pl.program_id(0)
    nsteps = pl.num_programs(0)
    xb = x_blk[...].astype(jnp.bfloat16)
    sc = jax.lax.dot_general(xb.reshape(B * BLK, D),
                             ws[...].astype(jnp.bfloat16),
                             (((1,), (1,)), ((), ())),
                             preferred_element_type=jnp.float32)
    scores_s[:, pl.ds(i * BLK, BLK)] = sc[:, 0].reshape(B, BLK)

    @pl.when(i == nsteps - 1)
    def _finish():
        iota = jax.lax.broadcasted_iota(jnp.int32, (1, S), 1)
        for b in range(B):
            row = scores_s[pl.ds(b, 1), :]
            m1 = jnp.max(row)
            i1 = jnp.min(jnp.where(row == m1, iota, S))
            row2 = jnp.where(iota == i1, NEG, row)
            m2 = jnp.max(row2)
            i2 = jnp.min(jnp.where(row2 == m2, iota, S))
            cp1 = pltpu.make_async_copy(
                x_any.at[b].at[pl.ds(i1, 1)], rows_s.at[pl.ds(2 * b, 1)], sem)
            cp1.start()
            cp2 = pltpu.make_async_copy(
                x_any.at[b].at[pl.ds(i2, 1)], rows_s.at[pl.ds(2 * b + 1, 1)], sem)
            cp2.start()
            cp1.wait()
            cp2.wait()
        xmean = rows_s[...].reshape(B, 2, D).mean(axis=1)
        summ = jax.lax.dot_general(xmean, wr[...], (((1,), (1,)), ((), ())),
                                   preferred_element_type=jnp.float32)
        summary_out[...] = summ + br[...]


def _ln_kernel(x_blk, summary, gamma, beta, out):
    h = x_blk[...] + summary[...][:, None, :]
    mu = jnp.mean(h, axis=-1, keepdims=True)
    hc = h - mu
    var = jnp.mean(hc * hc, axis=-1, keepdims=True)
    inv = jax.lax.rsqrt(var + 1e-5)
    out[...] = hc * inv * gamma[...][None, :, :] + beta[...][None, :, :]


@jax.jit
def kernel(x, Wr, br, Ws, bs, gamma, beta):
    del bs
    summary = pl.pallas_call(
        _scores_topk_summary_kernel,
        grid=(S // BLK,),
        in_specs=[
            pl.BlockSpec((B, BLK, D), lambda i: (0, i, 0)),
            pl.BlockSpec((8, D), lambda i: (0, 0)),
            pl.BlockSpec(memory_space=pl.ANY),
            pl.BlockSpec((D, D), lambda i: (0, 0)),
            pl.BlockSpec((1, D), lambda i: (0, 0)),
        ],
        out_specs=pl.BlockSpec((B, D), lambda i: (0, 0)),
        out_shape=jax.ShapeDtypeStruct((B, D), jnp.float32),
        scratch_shapes=[
            pltpu.VMEM((B, S), jnp.float32),
            pltpu.VMEM((2 * B, D), jnp.float32),
            pltpu.SemaphoreType.DMA,
        ],
        compiler_params=pltpu.CompilerParams(
            dimension_semantics=("arbitrary",)),
    )(x, jnp.broadcast_to(Ws, (8, D)), x, Wr, br.reshape(1, D))

    out = pl.pallas_call(
        _ln_kernel,
        grid=(S // BLKL,),
        in_specs=[
            pl.BlockSpec((B, BLKL, D), lambda i: (0, i, 0)),
            pl.BlockSpec((B, D), lambda i: (0, 0)),
            pl.BlockSpec((1, D), lambda i: (0, 0)),
            pl.BlockSpec((1, D), lambda i: (0, 0)),
        ],
        out_specs=pl.BlockSpec((B, BLKL, D), lambda i: (0, i, 0)),
        out_shape=jax.ShapeDtypeStruct((B, S, D), jnp.float32),
        compiler_params=pltpu.CompilerParams(
            dimension_semantics=("arbitrary",)),
    )(x, summary, gamma.reshape(1, D), beta.reshape(1, D))
    return out
